# baseline (speedup 1.0000x reference)
import functools

import jax
import jax.numpy as jnp
from jax import lax
from jax.experimental import pallas as pl
from jax.experimental.pallas import tpu as pltpu
from jax.experimental.pallas import tpu_sc as plsc

R = 128
C = 32768
L = 16
NC = 2
NS = 16
NW = NC * NS
ROWS_PER_W = R // NW
U = 8
STEPS = C // L

_mesh = plsc.VectorSubcoreMesh(core_axis_name="c", subcore_axis_name="s")


def _shuffle(x, idx):
    return lax.gather(
        x, idx[:, None],
        lax.GatherDimensionNumbers(
            offset_dims=(), collapsed_slice_dims=(0,), start_index_map=(0,)),
        slice_sizes=(1,),
        mode=lax.GatherScatterMode.PROMISE_IN_BOUNDS)


@functools.partial(
    pl.kernel,
    out_type=jax.ShapeDtypeStruct((NW, L), jnp.int32),
    mesh=_mesh,
    scratch_types=[
        pltpu.VMEM((C,), jnp.float32),
        pltpu.VMEM((C,), jnp.float32),
        pltpu.VMEM((L,), jnp.int32),
        pltpu.SemaphoreType.DMA,
        pltpu.SemaphoreType.DMA,
    ],
    compiler_params=pltpu.CompilerParams(needs_layout_passes=False),
)
def _row_argmax(data_hbm, idx_hbm, in0, in1, idx_v, sem0, sem1):
    wid = lax.axis_index("s") * NC + lax.axis_index("c")
    lanes = lax.iota(jnp.int32, L)
    bufs = (in0, in1)
    sems = (sem0, sem1)
    base_row = wid * ROWS_PER_W

    cps = [pltpu.async_copy(data_hbm.at[base_row], in0, sem0), None]
    acc = jnp.zeros((L,), jnp.int32)
    for r in range(ROWS_PER_W):
        cps[r % 2].wait()
        if r + 1 < ROWS_PER_W:
            cps[(r + 1) % 2] = pltpu.async_copy(
                data_hbm.at[base_row + r + 1], bufs[(r + 1) % 2],
                sems[(r + 1) % 2])
        buf = bufs[r % 2]

        def step(t, carry, buf=buf):
            bv, bi = carry
            base = t * (U * L)
            for k in range(U):
                v = buf[pl.ds(base + k * L, L)]
                idx = (base + k * L) + lanes
                upd = v > bv
                bv = jnp.where(upd, v, bv)
                bi = jnp.where(upd, idx, bi)
            return bv, bi

        init = (jnp.full((L,), -jnp.inf, jnp.float32),
                jnp.zeros((L,), jnp.int32))
        bv, bi = lax.fori_loop(0, STEPS // U, step, init)

        for k in (8, 4, 2, 1):
            pv = _shuffle(bv, lanes ^ k)
            pi = _shuffle(bi, lanes ^ k)
            take = (pv > bv) | ((pv == bv) & (pi < bi))
            bv = jnp.where(take, pv, bv)
            bi = jnp.where(take, pi, bi)

        acc = jnp.where(lanes == r, bi, acc)

    idx_v[...] = acc
    pltpu.sync_copy(idx_v, idx_hbm.at[wid])


_ZBLK = 16


def _zeros_body(out_ref):
    out_ref[...] = jnp.zeros((_ZBLK, C // 128, 128), jnp.float32)


_zeros = pl.pallas_call(
    _zeros_body,
    grid=(R // _ZBLK,),
    out_specs=pl.BlockSpec((_ZBLK, C // 128, 128), lambda i: (i, 0, 0)),
    out_shape=jax.ShapeDtypeStruct((R, C // 128, 128), jnp.float32),
)


_SBS = 8
_BLK_ELEMS = _SBS * 128


def _scatter_body(idx_sref, zeros_ref, out_ref):
    del zeros_ref
    i = pl.program_id(0)
    s = idx_sref[i // ROWS_PER_W, i % ROWS_PER_W]
    col_in_blk = s % _BLK_ELEMS
    io = (lax.broadcasted_iota(jnp.int32, (1, _SBS, 128), 1) * 128
          + lax.broadcasted_iota(jnp.int32, (1, _SBS, 128), 2))
    out_ref[...] = (io == col_in_blk).astype(jnp.float32)


_scatter = pl.pallas_call(
    _scatter_body,
    grid_spec=pltpu.PrefetchScalarGridSpec(
        num_scalar_prefetch=1,
        grid=(R,),
        in_specs=[pl.BlockSpec(memory_space=pl.ANY)],
        out_specs=pl.BlockSpec(
            (1, _SBS, 128),
            lambda i, idx_sref: (
                i, idx_sref[i // ROWS_PER_W, i % ROWS_PER_W] // _BLK_ELEMS, 0),
        ),
    ),
    out_shape=jax.ShapeDtypeStruct((R, C // 128, 128), jnp.float32),
    input_output_aliases={1: 0},
)


def kernel(data):
    z = _zeros()
    idx2d = _row_argmax(data)
    out3 = _scatter(idx2d, z)
    return out3.reshape(R, C)


# speedup vs baseline: 2.5237x; 2.5237x over previous
import functools

import jax
import jax.numpy as jnp
from jax import lax
from jax.experimental import pallas as pl
from jax.experimental.pallas import tpu as pltpu
from jax.experimental.pallas import tpu_sc as plsc

R = 128
C = 32768
L = 16
NC = 2
NS = 16
NW = NC * NS
ROWS_PER_W = R // NW
U = 8
STEPS = C // L

_mesh = plsc.VectorSubcoreMesh(core_axis_name="c", subcore_axis_name="s")


def _shuffle(x, idx):
    return lax.gather(
        x, idx[:, None],
        lax.GatherDimensionNumbers(
            offset_dims=(), collapsed_slice_dims=(0,), start_index_map=(0,)),
        slice_sizes=(1,),
        mode=lax.GatherScatterMode.PROMISE_IN_BOUNDS)


@functools.partial(
    pl.kernel,
    out_type=jax.ShapeDtypeStruct((R, C), jnp.float32),
    mesh=_mesh,
    scratch_types=[
        pltpu.VMEM((C,), jnp.float32),
        pltpu.VMEM((C,), jnp.float32),
        pltpu.VMEM((C,), jnp.float32),
        pltpu.SemaphoreType.DMA,
        pltpu.SemaphoreType.DMA,
        pltpu.SemaphoreType.DMA,
    ],
    compiler_params=pltpu.CompilerParams(needs_layout_passes=False),
)
def _argmax_onehot(data_hbm, out_hbm, in0, in1, out_v, sem0, sem1, sem_out):
    wid = lax.axis_index("s") * NC + lax.axis_index("c")
    lanes = lax.iota(jnp.int32, L)
    zeros = jnp.zeros((L,), jnp.float32)
    ones = jnp.ones((L,), jnp.float32)
    bufs = (in0, in1)
    sems = (sem0, sem1)
    base_row = wid * ROWS_PER_W

    cps = [pltpu.async_copy(data_hbm.at[base_row], in0, sem0), None]

    def zfill(t, _):
        base = t * (U * L)
        for k in range(U):
            out_v[pl.ds(base + k * L, L)] = zeros
        return 0

    lax.fori_loop(0, STEPS // U, zfill, 0)

    out_cp = None
    prev_bi = None
    for r in range(ROWS_PER_W):
        cps[r % 2].wait()
        if r + 1 < ROWS_PER_W:
            cps[(r + 1) % 2] = pltpu.async_copy(
                data_hbm.at[base_row + r + 1], bufs[(r + 1) % 2],
                sems[(r + 1) % 2])
        buf = bufs[r % 2]

        def step(t, carry, buf=buf):
            bv, bi = carry
            base = t * (U * L)
            for k in range(U):
                v = buf[pl.ds(base + k * L, L)]
                idx = (base + k * L) + lanes
                upd = v > bv
                bv = jnp.where(upd, v, bv)
                bi = jnp.where(upd, idx, bi)
            return bv, bi

        init = (jnp.full((L,), -jnp.inf, jnp.float32),
                jnp.zeros((L,), jnp.int32))
        bv, bi = lax.fori_loop(0, STEPS // U, step, init)

        for k in (8, 4, 2, 1):
            pv = _shuffle(bv, lanes ^ k)
            pi = _shuffle(bi, lanes ^ k)
            take = (pv > bv) | ((pv == bv) & (pi < bi))
            bv = jnp.where(take, pv, bv)
            bi = jnp.where(take, pi, bi)

        if out_cp is not None:
            out_cp.wait()
            plsc.store_scatter(out_v, [prev_bi], zeros, mask=lanes == 0)
        plsc.store_scatter(out_v, [bi], ones, mask=lanes == 0)
        out_cp = pltpu.async_copy(out_v, out_hbm.at[base_row + r], sem_out)
        prev_bi = bi

    out_cp.wait()


def kernel(data):
    return _argmax_onehot(data)
